# Initial kernel scaffold; baseline (speedup 1.0000x reference)
import jax
import jax.numpy as jnp
from jax.experimental import pallas as pl


def kernel(x, edge_index, edge_attr, W1, b1, W2, b2, W3, b3, W4, b4, U1, c1, U2, c2, U3, c3):
    raise NotImplementedError("write your pallas kernel here")


# speedup vs baseline: 3.2775x; 3.2775x over previous
import functools

import jax
import jax.numpy as jnp
from jax import lax
from jax.experimental import pallas as pl
from jax.experimental.pallas import tpu as pltpu
from jax.experimental.pallas import tpu_sc as plsc

N_NODES = 100000
NF = 6
H = 16

NC = 2
NS = 16
NW = NC * NS

IDX_ROW = 128
ROWS_PER_CHUNK = 16
CHUNK = IDX_ROW * ROWS_PER_CHUNK
EDGES_PER_SUPER = NW * CHUNK

HALF = 50176
SLAB = HALF // NS
AGG_ROWS = HALF + IDX_ROW

_mesh = lambda: plsc.VectorSubcoreMesh(core_axis_name="c", subcore_axis_name="s")


def _make_gather(E_pad):
    rows_total = E_pad // IDX_ROW
    rows_per_w = rows_total // NW
    chunks = rows_per_w // ROWS_PER_CHUNK

    @functools.partial(
        pl.kernel,
        out_type=(jax.ShapeDtypeStruct((E_pad, H), jnp.float32),
                  jax.ShapeDtypeStruct((E_pad, H), jnp.float32)),
        mesh=_mesh(),
        compiler_params=pltpu.CompilerParams(use_tc_tiling_on_sc=False),
        scratch_types=[
            pltpu.VMEM((ROWS_PER_CHUNK, IDX_ROW), jnp.int32),
            pltpu.VMEM((ROWS_PER_CHUNK, IDX_ROW), jnp.int32),
            pltpu.VMEM((CHUNK, H), jnp.float32),
            pltpu.VMEM((CHUNK, H), jnp.float32),
            pltpu.SemaphoreType.DMA,
        ],
    )
    def gather(psrc, pdst, src2d, dst2d, a_out, b_out, sidx, didx, av, bv, sem):
        wid = lax.axis_index("s") * NC + lax.axis_index("c")
        wrow = wid * rows_per_w

        def body(g, carry):
            row0 = wrow + g * ROWS_PER_CHUNK
            pltpu.sync_copy(src2d.at[pl.ds(row0, ROWS_PER_CHUNK)], sidx)
            pltpu.sync_copy(dst2d.at[pl.ds(row0, ROWS_PER_CHUNK)], didx)
            cps = []
            for j in range(ROWS_PER_CHUNK):
                cps.append(pltpu.async_copy(
                    psrc.at[sidx.at[j]], av.at[pl.ds(j * IDX_ROW, IDX_ROW)], sem))
                cps.append(pltpu.async_copy(
                    pdst.at[didx.at[j]], bv.at[pl.ds(j * IDX_ROW, IDX_ROW)], sem))
            for cp in cps:
                cp.wait()
            e0 = row0 * IDX_ROW
            pltpu.sync_copy(av, a_out.at[pl.ds(e0, CHUNK)])
            pltpu.sync_copy(bv, b_out.at[pl.ds(e0, CHUNK)])
            return carry

        lax.fori_loop(0, chunks, body, 0)

    return gather


def _make_scatter(E_pad):
    rows_total = E_pad // IDX_ROW
    rows_per_t = rows_total // NS
    chunks = rows_per_t // ROWS_PER_CHUNK

    @functools.partial(
        pl.kernel,
        out_type=jax.ShapeDtypeStruct((NC * HALF, H), jnp.float32),
        mesh=_mesh(),
        compiler_params=pltpu.CompilerParams(use_tc_tiling_on_sc=False),
        scratch_types=[
            pltpu.VMEM((ROWS_PER_CHUNK, IDX_ROW), jnp.int32),
            pltpu.VMEM((CHUNK, H), jnp.float32),
            pltpu.VMEM_SHARED((AGG_ROWS, H), jnp.float32),
            pltpu.SemaphoreType.DMA,
        ],
    )
    def scatter(msg, lidx2d, zeros, out, didx, mv, aggr, sem):
        c = lax.axis_index("c")
        s = lax.axis_index("s")

        pltpu.sync_copy(zeros, mv)
        for t in range(2):
            pltpu.sync_copy(mv.at[pl.ds(0, SLAB // 2)],
                            aggr.at[pl.ds(s * SLAB + t * (SLAB // 2), SLAB // 2)])

        @pl.when(s == 0)
        def _():
            pltpu.sync_copy(mv.at[pl.ds(0, IDX_ROW)],
                            aggr.at[pl.ds(HALF, IDX_ROW)])

        plsc.subcore_barrier()

        trow = s * rows_per_t

        def body(g, carry):
            row0 = trow + g * ROWS_PER_CHUNK
            pltpu.sync_copy(lidx2d.at[pl.ds(c * rows_total + row0,
                                            ROWS_PER_CHUNK)], didx)
            pltpu.sync_copy(msg.at[pl.ds(row0 * IDX_ROW, CHUNK)], mv)
            for j in range(ROWS_PER_CHUNK):
                pltpu.sync_copy(mv.at[pl.ds(j * IDX_ROW, IDX_ROW)],
                                aggr.at[didx.at[j]], add=True)
            return carry

        lax.fori_loop(0, chunks, body, 0)
        plsc.subcore_barrier()

        pltpu.sync_copy(aggr.at[pl.ds(s * SLAB, SLAB)],
                        out.at[pl.ds(c * HALF + s * SLAB, SLAB)])

    return scatter


def _node_proj_body(xr, w1a_r, w1b_r, b1_r, ps_r, pd_r):
    xb = xr[...]
    ps_r[...] = jnp.dot(xb, w1a_r[...], preferred_element_type=jnp.float32)
    pd_r[...] = jnp.dot(xb, w1b_r[...], preferred_element_type=jnp.float32) + b1_r[...]


def _edge_mlp_body(ar, br, er, dr_idx, w1c_r, w2_r, b2_r, w3_r, b3_r, w4_r,
                   b4_r, mr, lr):
    t = ar[...] + br[...] + jnp.dot(er[...], w1c_r[...],
                                    preferred_element_type=jnp.float32)
    h = jnp.tanh(t)
    h = jnp.tanh(jnp.dot(h, w2_r[...], preferred_element_type=jnp.float32) + b2_r[...])
    h = jnp.tanh(jnp.dot(h, w3_r[...], preferred_element_type=jnp.float32) + b3_r[...])
    mr[...] = jnp.dot(h, w4_r[...], preferred_element_type=jnp.float32) + b4_r[...]
    d = dr_idx[...]
    lr[0, ...] = jnp.where(d < HALF, d, HALF)
    d1 = d - HALF
    lr[1, ...] = jnp.where(d1 >= 0, d1, HALF)


def _update_body(xr, p0_r, u1x_r, u1a_r, c1_r, u2_r, c2_r, u3_r, c3_r, dr):
    aggr = p0_r[...]
    u = jnp.tanh(jnp.dot(xr[...], u1x_r[...], preferred_element_type=jnp.float32)
                 + jnp.dot(aggr, u1a_r[...], preferred_element_type=jnp.float32)
                 + c1_r[...])
    u = jnp.tanh(jnp.dot(u, u2_r[...], preferred_element_type=jnp.float32) + c2_r[...])
    dr[...] = jnp.dot(u, u3_r[...], preferred_element_type=jnp.float32) + c3_r[...]


def _full_spec(shape):
    return pl.BlockSpec(shape, lambda i: (0,) * len(shape))


def kernel(x, edge_index, edge_attr, W1, b1, W2, b2, W3, b3, W4, b4,
           U1, c1, U2, c2, U3, c3):
    E = edge_index.shape[1]
    E_pad = -(-E // EDGES_PER_SUPER) * EDGES_PER_SUPER
    pad = E_pad - E

    src = edge_index[0]
    dst = edge_index[1]
    src2d = jnp.pad(src, (0, pad)).reshape(-1, IDX_ROW)
    dst2d = jnp.pad(dst, (0, pad), constant_values=N_NODES).reshape(-1, IDX_ROW)
    ea = jnp.pad(edge_attr, ((0, pad), (0, 0)))

    RN = 2000
    gn = N_NODES // RN
    psrc, pdst = pl.pallas_call(
        _node_proj_body,
        grid=(gn,),
        in_specs=[
            pl.BlockSpec((RN, NF), lambda i: (i, 0)),
            _full_spec((NF, H)), _full_spec((NF, H)), _full_spec((1, H)),
        ],
        out_specs=[pl.BlockSpec((RN, H), lambda i: (i, 0)),
                   pl.BlockSpec((RN, H), lambda i: (i, 0))],
        out_shape=[jax.ShapeDtypeStruct((N_NODES, H), jnp.float32),
                   jax.ShapeDtypeStruct((N_NODES, H), jnp.float32)],
    )(x, W1[NF:2 * NF], W1[:NF], b1.reshape(1, H))

    a_g, b_g = _make_gather(E_pad)(psrc, pdst, src2d, dst2d)

    RE = 8192
    RROW = RE // IDX_ROW
    ge = E_pad // RE
    rows_total = E_pad // IDX_ROW
    msg, lidx = pl.pallas_call(
        _edge_mlp_body,
        grid=(ge,),
        in_specs=[
            pl.BlockSpec((RE, H), lambda i: (i, 0)),
            pl.BlockSpec((RE, H), lambda i: (i, 0)),
            pl.BlockSpec((RE, 3), lambda i: (i, 0)),
            pl.BlockSpec((RROW, IDX_ROW), lambda i: (i, 0)),
            _full_spec((3, H)),
            _full_spec((H, H)), _full_spec((1, H)),
            _full_spec((H, H)), _full_spec((1, H)),
            _full_spec((H, H)), _full_spec((1, H)),
        ],
        out_specs=[pl.BlockSpec((RE, H), lambda i: (i, 0)),
                   pl.BlockSpec((NC, RROW, IDX_ROW), lambda i: (0, i, 0))],
        out_shape=[jax.ShapeDtypeStruct((E_pad, H), jnp.float32),
                   jax.ShapeDtypeStruct((NC, rows_total, IDX_ROW), jnp.int32)],
    )(a_g, b_g, ea, dst2d, W1[2 * NF:], W2, b2.reshape(1, H), W3,
      b3.reshape(1, H), W4, b4.reshape(1, H))

    zeros = jnp.zeros((CHUNK, H), jnp.float32)
    aggr = _make_scatter(E_pad)(msg, lidx.reshape(NC * rows_total, IDX_ROW),
                                zeros)

    delta = pl.pallas_call(
        _update_body,
        grid=(gn,),
        in_specs=[
            pl.BlockSpec((RN, NF), lambda i: (i, 0)),
            pl.BlockSpec((RN, H), lambda i: (i, 0)),
            _full_spec((NF, H)), _full_spec((H, H)), _full_spec((1, H)),
            _full_spec((H, H)), _full_spec((1, H)),
            _full_spec((H, 3)), _full_spec((1, 3)),
        ],
        out_specs=pl.BlockSpec((RN, 3), lambda i: (i, 0)),
        out_shape=jax.ShapeDtypeStruct((N_NODES, 3), jnp.float32),
    )(x, aggr[:N_NODES], U1[:NF], U1[NF:],
      c1.reshape(1, H), U2, c2.reshape(1, H), U3, c3.reshape(1, 3))

    return delta


# speedup vs baseline: 4.7179x; 1.4395x over previous
import functools

import jax
import jax.numpy as jnp
from jax import lax
from jax.experimental import pallas as pl
from jax.experimental.pallas import tpu as pltpu
from jax.experimental.pallas import tpu_sc as plsc

N_NODES = 100000
NF = 6
H = 16

NC = 2
NS = 16
NW = NC * NS

IDX_ROW = 128
ROWS_PER_CHUNK = 16
CHUNK = IDX_ROW * ROWS_PER_CHUNK
EDGES_PER_SUPER = NW * CHUNK

HALF = 50176
SLAB = HALF // NS
AGG_ROWS = HALF + IDX_ROW

_mesh = lambda: plsc.VectorSubcoreMesh(core_axis_name="c", subcore_axis_name="s")


def _make_gather(E_pad):
    rows_total = E_pad // IDX_ROW
    total_chunks = rows_total // ROWS_PER_CHUNK
    trips = -(-total_chunks // NW)

    @functools.partial(
        pl.kernel,
        out_type=(jax.ShapeDtypeStruct((E_pad, H), jnp.float32),
                  jax.ShapeDtypeStruct((E_pad, H), jnp.float32)),
        mesh=_mesh(),
        compiler_params=pltpu.CompilerParams(use_tc_tiling_on_sc=False),
        scratch_types=[
            pltpu.VMEM((ROWS_PER_CHUNK, IDX_ROW), jnp.int32),
            pltpu.VMEM((ROWS_PER_CHUNK, IDX_ROW), jnp.int32),
            pltpu.VMEM((CHUNK, H), jnp.float32),
            pltpu.VMEM((CHUNK, H), jnp.float32),
            pltpu.SemaphoreType.DMA,
        ],
    )
    def gather(psrc, pdst, src2d, dst2d, a_out, b_out, sidx, didx, av, bv, sem):
        wid = lax.axis_index("s") * NC + lax.axis_index("c")

        def body(g, carry):
            ck = wid + g * NW

            @pl.when(ck < total_chunks)
            def _():
                row0 = ck * ROWS_PER_CHUNK
                pltpu.sync_copy(src2d.at[pl.ds(row0, ROWS_PER_CHUNK)], sidx)
                pltpu.sync_copy(dst2d.at[pl.ds(row0, ROWS_PER_CHUNK)], didx)
                cps = []
                for j in range(ROWS_PER_CHUNK):
                    cps.append(pltpu.async_copy(
                        psrc.at[sidx.at[j]], av.at[pl.ds(j * IDX_ROW, IDX_ROW)], sem))
                    cps.append(pltpu.async_copy(
                        pdst.at[didx.at[j]], bv.at[pl.ds(j * IDX_ROW, IDX_ROW)], sem))
                for cp in cps:
                    cp.wait()
                e0 = row0 * IDX_ROW
                pltpu.sync_copy(av, a_out.at[pl.ds(e0, CHUNK)])
                pltpu.sync_copy(bv, b_out.at[pl.ds(e0, CHUNK)])

            return carry

        lax.fori_loop(0, trips, body, 0)

    return gather


def _make_scatter(E_pad):
    rows_total = E_pad // IDX_ROW
    total_chunks = rows_total // ROWS_PER_CHUNK
    trips = -(-total_chunks // NS)

    @functools.partial(
        pl.kernel,
        out_type=jax.ShapeDtypeStruct((NC * HALF, H), jnp.float32),
        mesh=_mesh(),
        compiler_params=pltpu.CompilerParams(use_tc_tiling_on_sc=False),
        scratch_types=[
            pltpu.VMEM((ROWS_PER_CHUNK, IDX_ROW), jnp.int32),
            pltpu.VMEM((CHUNK, H), jnp.float32),
            pltpu.VMEM_SHARED((AGG_ROWS, H), jnp.float32),
            pltpu.SemaphoreType.DMA,
        ],
    )
    def scatter(msg, lidx2d, zeros, out, didx, mv, aggr, sem):
        c = lax.axis_index("c")
        s = lax.axis_index("s")

        pltpu.sync_copy(zeros, mv)
        for t in range(2):
            pltpu.sync_copy(mv.at[pl.ds(0, SLAB // 2)],
                            aggr.at[pl.ds(s * SLAB + t * (SLAB // 2), SLAB // 2)])

        @pl.when(s == 0)
        def _():
            pltpu.sync_copy(mv.at[pl.ds(0, IDX_ROW)],
                            aggr.at[pl.ds(HALF, IDX_ROW)])

        plsc.subcore_barrier()

        def body(g, carry):
            ck = s + g * NS

            @pl.when(ck < total_chunks)
            def _():
                row0 = ck * ROWS_PER_CHUNK
                pltpu.sync_copy(lidx2d.at[pl.ds(c * rows_total + row0,
                                                ROWS_PER_CHUNK)], didx)
                pltpu.sync_copy(msg.at[pl.ds(row0 * IDX_ROW, CHUNK)], mv)
                for j in range(ROWS_PER_CHUNK):
                    pltpu.sync_copy(mv.at[pl.ds(j * IDX_ROW, IDX_ROW)],
                                    aggr.at[didx.at[j]], add=True)

            return carry

        lax.fori_loop(0, trips, body, 0)
        plsc.subcore_barrier()

        pltpu.sync_copy(aggr.at[pl.ds(s * SLAB, SLAB)],
                        out.at[pl.ds(c * HALF + s * SLAB, SLAB)])

    return scatter


def _node_proj_body(xr, w1a_r, w1b_r, b1_r, ps_r, pd_r):
    xb = xr[...]
    ps_r[...] = jnp.dot(xb, w1a_r[...], preferred_element_type=jnp.float32)
    pd_r[...] = jnp.dot(xb, w1b_r[...], preferred_element_type=jnp.float32) + b1_r[...]


def _edge_mlp_body(ar, br, er, dr_idx, w1c_r, w2_r, b2_r, w3_r, b3_r, w4_r,
                   b4_r, mr, lr):
    t = ar[...] + br[...] + jnp.dot(er[...], w1c_r[...],
                                    preferred_element_type=jnp.float32)
    h = jnp.tanh(t)
    h = jnp.tanh(jnp.dot(h, w2_r[...], preferred_element_type=jnp.float32) + b2_r[...])
    h = jnp.tanh(jnp.dot(h, w3_r[...], preferred_element_type=jnp.float32) + b3_r[...])
    mr[...] = jnp.dot(h, w4_r[...], preferred_element_type=jnp.float32) + b4_r[...]
    d = dr_idx[...]
    lr[0, ...] = jnp.where(d < HALF, d, HALF)
    d1 = d - HALF
    lr[1, ...] = jnp.where(d1 >= 0, d1, HALF)


def _update_body(xr, p0_r, u1x_r, u1a_r, c1_r, u2_r, c2_r, u3_r, c3_r, dr):
    aggr = p0_r[...]
    u = jnp.tanh(jnp.dot(xr[...], u1x_r[...], preferred_element_type=jnp.float32)
                 + jnp.dot(aggr, u1a_r[...], preferred_element_type=jnp.float32)
                 + c1_r[...])
    u = jnp.tanh(jnp.dot(u, u2_r[...], preferred_element_type=jnp.float32) + c2_r[...])
    dr[...] = jnp.dot(u, u3_r[...], preferred_element_type=jnp.float32) + c3_r[...]


def _full_spec(shape):
    return pl.BlockSpec(shape, lambda i: (0,) * len(shape))


def kernel(x, edge_index, edge_attr, W1, b1, W2, b2, W3, b3, W4, b4,
           U1, c1, U2, c2, U3, c3):
    E = edge_index.shape[1]
    assert E % CHUNK == 0, "edge count must be a multiple of 2048"
    E_pad = E

    src2d = edge_index[0].reshape(-1, IDX_ROW)
    dst2d = edge_index[1].reshape(-1, IDX_ROW)
    ea = edge_attr

    RN = 2000
    gn = N_NODES // RN
    psrc, pdst = pl.pallas_call(
        _node_proj_body,
        grid=(gn,),
        in_specs=[
            pl.BlockSpec((RN, NF), lambda i: (i, 0)),
            _full_spec((NF, H)), _full_spec((NF, H)), _full_spec((1, H)),
        ],
        out_specs=[pl.BlockSpec((RN, H), lambda i: (i, 0)),
                   pl.BlockSpec((RN, H), lambda i: (i, 0))],
        out_shape=[jax.ShapeDtypeStruct((N_NODES, H), jnp.float32),
                   jax.ShapeDtypeStruct((N_NODES, H), jnp.float32)],
    )(x, W1[NF:2 * NF], W1[:NF], b1.reshape(1, H))

    a_g, b_g = _make_gather(E_pad)(psrc, pdst, src2d, dst2d)

    RE = 10240
    RROW = RE // IDX_ROW
    ge = E_pad // RE
    rows_total = E_pad // IDX_ROW
    msg, lidx = pl.pallas_call(
        _edge_mlp_body,
        grid=(ge,),
        in_specs=[
            pl.BlockSpec((RE, H), lambda i: (i, 0)),
            pl.BlockSpec((RE, H), lambda i: (i, 0)),
            pl.BlockSpec((RE, 3), lambda i: (i, 0)),
            pl.BlockSpec((RROW, IDX_ROW), lambda i: (i, 0)),
            _full_spec((3, H)),
            _full_spec((H, H)), _full_spec((1, H)),
            _full_spec((H, H)), _full_spec((1, H)),
            _full_spec((H, H)), _full_spec((1, H)),
        ],
        out_specs=[pl.BlockSpec((RE, H), lambda i: (i, 0)),
                   pl.BlockSpec((NC, RROW, IDX_ROW), lambda i: (0, i, 0))],
        out_shape=[jax.ShapeDtypeStruct((E_pad, H), jnp.float32),
                   jax.ShapeDtypeStruct((NC, rows_total, IDX_ROW), jnp.int32)],
    )(a_g, b_g, ea, dst2d, W1[2 * NF:], W2, b2.reshape(1, H), W3,
      b3.reshape(1, H), W4, b4.reshape(1, H))

    zeros = jnp.zeros((CHUNK, H), jnp.float32)
    aggr = _make_scatter(E_pad)(msg, lidx.reshape(NC * rows_total, IDX_ROW),
                                zeros)

    delta = pl.pallas_call(
        _update_body,
        grid=(gn,),
        in_specs=[
            pl.BlockSpec((RN, NF), lambda i: (i, 0)),
            pl.BlockSpec((RN, H), lambda i: (i, 0)),
            _full_spec((NF, H)), _full_spec((H, H)), _full_spec((1, H)),
            _full_spec((H, H)), _full_spec((1, H)),
            _full_spec((H, 3)), _full_spec((1, 3)),
        ],
        out_specs=pl.BlockSpec((RN, 3), lambda i: (i, 0)),
        out_shape=jax.ShapeDtypeStruct((N_NODES, 3), jnp.float32),
    )(x, aggr, U1[:NF], U1[NF:],
      c1.reshape(1, H), U2, c2.reshape(1, H), U3, c3.reshape(1, 3))

    return delta


# speedup vs baseline: 5.7800x; 1.2251x over previous
import functools

import jax
import jax.numpy as jnp
from jax import lax
from jax.experimental import pallas as pl
from jax.experimental.pallas import tpu as pltpu
from jax.experimental.pallas import tpu_sc as plsc

N_NODES = 100000
NF = 6
H = 16

NC = 2
NS = 16
NW = NC * NS

IDX_ROW = 128
ROWS_PER_CHUNK = 16
CHUNK = IDX_ROW * ROWS_PER_CHUNK
EDGES_PER_SUPER = NW * CHUNK

HALF = 50176
SLAB = HALF // NS
AGG_ROWS = HALF + IDX_ROW

_mesh = lambda: plsc.VectorSubcoreMesh(core_axis_name="c", subcore_axis_name="s")


def _make_gather(E_pad):
    rows_total = E_pad // IDX_ROW
    total_chunks = rows_total // ROWS_PER_CHUNK
    trips = -(-total_chunks // NW)

    @functools.partial(
        pl.kernel,
        out_type=(jax.ShapeDtypeStruct((E_pad, H), jnp.float32),
                  jax.ShapeDtypeStruct((E_pad, H), jnp.float32)),
        mesh=_mesh(),
        compiler_params=pltpu.CompilerParams(use_tc_tiling_on_sc=False),
        scratch_types=[
            pltpu.VMEM((ROWS_PER_CHUNK, IDX_ROW), jnp.int32),
            pltpu.VMEM((ROWS_PER_CHUNK, IDX_ROW), jnp.int32),
            pltpu.VMEM((CHUNK, H), jnp.float32),
            pltpu.VMEM((CHUNK, H), jnp.float32),
            pltpu.SemaphoreType.DMA,
        ],
    )
    def gather(psrc, pdst, src2d, dst2d, a_out, b_out, sidx, didx, av, bv, sem):
        wid = lax.axis_index("s") * NC + lax.axis_index("c")

        def body(g, carry):
            ck = wid + g * NW

            @pl.when(ck < total_chunks)
            def _():
                row0 = ck * ROWS_PER_CHUNK
                pltpu.sync_copy(src2d.at[pl.ds(row0, ROWS_PER_CHUNK)], sidx)
                pltpu.sync_copy(dst2d.at[pl.ds(row0, ROWS_PER_CHUNK)], didx)
                cps = []
                for j in range(ROWS_PER_CHUNK):
                    cps.append(pltpu.async_copy(
                        psrc.at[sidx.at[j]], av.at[pl.ds(j * IDX_ROW, IDX_ROW)], sem))
                    cps.append(pltpu.async_copy(
                        pdst.at[didx.at[j]], bv.at[pl.ds(j * IDX_ROW, IDX_ROW)], sem))
                for cp in cps:
                    cp.wait()
                e0 = row0 * IDX_ROW
                pltpu.sync_copy(av, a_out.at[pl.ds(e0, CHUNK)])
                pltpu.sync_copy(bv, b_out.at[pl.ds(e0, CHUNK)])

            return carry

        lax.fori_loop(0, trips, body, 0)

    return gather


def _make_scatter(E_pad):
    rows_total = E_pad // IDX_ROW
    total_chunks = rows_total // ROWS_PER_CHUNK
    trips = -(-total_chunks // NS)

    @functools.partial(
        pl.kernel,
        out_type=jax.ShapeDtypeStruct((NC * HALF, H), jnp.float32),
        mesh=_mesh(),
        compiler_params=pltpu.CompilerParams(use_tc_tiling_on_sc=False),
        scratch_types=[
            pltpu.VMEM((ROWS_PER_CHUNK, IDX_ROW), jnp.int32),
            pltpu.VMEM((CHUNK, H), jnp.float32),
            pltpu.VMEM_SHARED((AGG_ROWS, H), jnp.float32),
            pltpu.SemaphoreType.DMA,
        ],
    )
    def scatter(msg, lidx2d, zeros, out, didx, mv, aggr, sem):
        c = lax.axis_index("c")
        s = lax.axis_index("s")

        pltpu.sync_copy(zeros, mv)
        for t in range(2):
            pltpu.sync_copy(mv.at[pl.ds(0, SLAB // 2)],
                            aggr.at[pl.ds(s * SLAB + t * (SLAB // 2), SLAB // 2)])

        @pl.when(s == 0)
        def _():
            pltpu.sync_copy(mv.at[pl.ds(0, IDX_ROW)],
                            aggr.at[pl.ds(HALF, IDX_ROW)])

        plsc.subcore_barrier()

        def body(g, carry):
            ck = s + g * NS

            @pl.when(ck < total_chunks)
            def _():
                row0 = ck * ROWS_PER_CHUNK
                pltpu.sync_copy(lidx2d.at[pl.ds(c * rows_total + row0,
                                                ROWS_PER_CHUNK)], didx)
                pltpu.sync_copy(msg.at[pl.ds(row0 * IDX_ROW, CHUNK)], mv)
                for j in range(ROWS_PER_CHUNK):
                    pltpu.sync_copy(mv.at[pl.ds(j * IDX_ROW, IDX_ROW)],
                                    aggr.at[didx.at[j]], add=True)

            return carry

        lax.fori_loop(0, trips, body, 0)
        plsc.subcore_barrier()

        pltpu.sync_copy(aggr.at[pl.ds(s * SLAB, SLAB)],
                        out.at[pl.ds(c * HALF + s * SLAB, SLAB)])

    return scatter


def _node_proj_body(xr, w1a_r, w1b_r, b1_r, ps_r, pd_r):
    xb = xr[...]
    ps_r[...] = jnp.dot(xb, w1a_r[...], preferred_element_type=jnp.float32)
    pd_r[...] = jnp.dot(xb, w1b_r[...], preferred_element_type=jnp.float32) + b1_r[...]


def _edge_mlp_body(ar, br, er, dr_idx, w1c_r, w2_r, b2_r, w3_r, b3_r, w4_r,
                   b4_r, mr, lr):
    t = ar[...] + br[...] + jnp.dot(er[...], w1c_r[...],
                                    preferred_element_type=jnp.float32)
    h = jnp.tanh(t)
    h = jnp.tanh(jnp.dot(h, w2_r[...], preferred_element_type=jnp.float32) + b2_r[...])
    h = jnp.tanh(jnp.dot(h, w3_r[...], preferred_element_type=jnp.float32) + b3_r[...])
    mr[...] = jnp.dot(h, w4_r[...], preferred_element_type=jnp.float32) + b4_r[...]
    d = dr_idx[...]
    lr[0, ...] = jnp.where(d < HALF, d, HALF)
    d1 = d - HALF
    lr[1, ...] = jnp.where(d1 >= 0, d1, HALF)


def _update_body(xr, p0_r, u1x_r, u1a_r, c1_r, u2_r, c2_r, u3_r, c3_r, dr):
    aggr = p0_r[...]
    u = jnp.tanh(jnp.dot(xr[...], u1x_r[...], preferred_element_type=jnp.float32)
                 + jnp.dot(aggr, u1a_r[...], preferred_element_type=jnp.float32)
                 + c1_r[...])
    u = jnp.tanh(jnp.dot(u, u2_r[...], preferred_element_type=jnp.float32) + c2_r[...])
    dr[...] = jnp.dot(u, u3_r[...], preferred_element_type=jnp.float32) + c3_r[...]


def _full_spec(shape):
    return pl.BlockSpec(shape, lambda i: (0,) * len(shape))


def kernel(x, edge_index, edge_attr, W1, b1, W2, b2, W3, b3, W4, b4,
           U1, c1, U2, c2, U3, c3):
    E = edge_index.shape[1]
    assert E % CHUNK == 0, "edge count must be a multiple of 2048"
    E_pad = E

    src2d = edge_index[0].reshape(-1, IDX_ROW)
    dst2d = edge_index[1].reshape(-1, IDX_ROW)

    eye8 = jnp.eye(8, dtype=jnp.float32)
    kron = lambda w: jnp.kron(eye8, w)
    tile8 = lambda b: jnp.tile(b, 8).reshape(1, 8 * b.shape[0])

    NP = N_NODES // 8
    psrc, pdst = pl.pallas_call(
        _node_proj_body,
        grid=(1,),
        in_specs=[
            pl.BlockSpec((NP, 8 * NF), lambda i: (0, 0)),
            _full_spec((8 * NF, 8 * H)), _full_spec((8 * NF, 8 * H)),
            _full_spec((1, 8 * H)),
        ],
        out_specs=[pl.BlockSpec((NP, 8 * H), lambda i: (0, 0)),
                   pl.BlockSpec((NP, 8 * H), lambda i: (0, 0))],
        out_shape=[jax.ShapeDtypeStruct((NP, 8 * H), jnp.float32),
                   jax.ShapeDtypeStruct((NP, 8 * H), jnp.float32)],
    )(x.reshape(NP, 8 * NF), kron(W1[NF:2 * NF]), kron(W1[:NF]), tile8(b1))

    a_g, b_g = _make_gather(E_pad)(psrc.reshape(N_NODES, H),
                                   pdst.reshape(N_NODES, H), src2d, dst2d)

    EP = E_pad // 8
    RE8 = 3200
    ge = EP // RE8
    RROW = 8 * RE8 // IDX_ROW
    rows_total = E_pad // IDX_ROW
    msg, lidx = pl.pallas_call(
        _edge_mlp_body,
        grid=(ge,),
        in_specs=[
            pl.BlockSpec((RE8, 8 * H), lambda i: (i, 0)),
            pl.BlockSpec((RE8, 8 * H), lambda i: (i, 0)),
            pl.BlockSpec((RE8, 24), lambda i: (i, 0)),
            pl.BlockSpec((RROW, IDX_ROW), lambda i: (i, 0)),
            _full_spec((24, 8 * H)),
            _full_spec((8 * H, 8 * H)), _full_spec((1, 8 * H)),
            _full_spec((8 * H, 8 * H)), _full_spec((1, 8 * H)),
            _full_spec((8 * H, 8 * H)), _full_spec((1, 8 * H)),
        ],
        out_specs=[pl.BlockSpec((RE8, 8 * H), lambda i: (i, 0)),
                   pl.BlockSpec((NC, RROW, IDX_ROW), lambda i: (0, i, 0))],
        out_shape=[jax.ShapeDtypeStruct((EP, 8 * H), jnp.float32),
                   jax.ShapeDtypeStruct((NC, rows_total, IDX_ROW), jnp.int32)],
    )(a_g.reshape(EP, 8 * H), b_g.reshape(EP, 8 * H),
      edge_attr.reshape(EP, 24), dst2d, kron(W1[2 * NF:]), kron(W2),
      tile8(b2), kron(W3), tile8(b3), kron(W4), tile8(b4))
    msg = msg.reshape(E_pad, H)

    zeros = jnp.zeros((CHUNK, H), jnp.float32)
    aggr = _make_scatter(E_pad)(msg, lidx.reshape(NC * rows_total, IDX_ROW),
                                zeros)

    RN = 2000
    gn = N_NODES // RN
    delta = pl.pallas_call(
        _update_body,
        grid=(gn,),
        in_specs=[
            pl.BlockSpec((RN, NF), lambda i: (i, 0)),
            pl.BlockSpec((RN, H), lambda i: (i, 0)),
            _full_spec((NF, H)), _full_spec((H, H)), _full_spec((1, H)),
            _full_spec((H, H)), _full_spec((1, H)),
            _full_spec((H, 3)), _full_spec((1, 3)),
        ],
        out_specs=pl.BlockSpec((RN, 3), lambda i: (i, 0)),
        out_shape=jax.ShapeDtypeStruct((N_NODES, 3), jnp.float32),
    )(x, aggr, U1[:NF], U1[NF:],
      c1.reshape(1, H), U2, c2.reshape(1, H), U3, c3.reshape(1, 3))

    return delta


# speedup vs baseline: 5.8017x; 1.0038x over previous
import functools

import jax
import jax.numpy as jnp
from jax import lax
from jax.experimental import pallas as pl
from jax.experimental.pallas import tpu as pltpu
from jax.experimental.pallas import tpu_sc as plsc

N_NODES = 100000
NF = 6
H = 16

NC = 2
NS = 16
NW = NC * NS

IDX_ROW = 128
ROWS_PER_CHUNK = 16
CHUNK = IDX_ROW * ROWS_PER_CHUNK
EDGES_PER_SUPER = NW * CHUNK

HALF = 50176
SLAB = HALF // NS
AGG_ROWS = HALF + IDX_ROW

_mesh = lambda: plsc.VectorSubcoreMesh(core_axis_name="c", subcore_axis_name="s")


def _make_gather(E_pad):
    rows_total = E_pad // IDX_ROW
    total_chunks = rows_total // ROWS_PER_CHUNK
    trips = -(-total_chunks // NW)

    @functools.partial(
        pl.kernel,
        out_type=(jax.ShapeDtypeStruct((E_pad, H), jnp.float32),
                  jax.ShapeDtypeStruct((E_pad, H), jnp.float32)),
        mesh=_mesh(),
        compiler_params=pltpu.CompilerParams(use_tc_tiling_on_sc=False),
        scratch_types=[
            pltpu.VMEM((ROWS_PER_CHUNK, IDX_ROW), jnp.int32),
            pltpu.VMEM((ROWS_PER_CHUNK, IDX_ROW), jnp.int32),
            pltpu.VMEM((CHUNK, H), jnp.float32),
            pltpu.VMEM((CHUNK, H), jnp.float32),
            pltpu.SemaphoreType.DMA,
        ],
    )
    def gather(psrc, pdst, src2d, dst2d, a_out, b_out, sidx, didx, av, bv, sem):
        wid = lax.axis_index("s") * NC + lax.axis_index("c")

        def body(g, carry):
            ck = wid + g * NW

            @pl.when(ck < total_chunks)
            def _():
                row0 = ck * ROWS_PER_CHUNK
                pltpu.sync_copy(src2d.at[pl.ds(row0, ROWS_PER_CHUNK)], sidx)
                pltpu.sync_copy(dst2d.at[pl.ds(row0, ROWS_PER_CHUNK)], didx)
                cps = []
                for j in range(ROWS_PER_CHUNK):
                    cps.append(pltpu.async_copy(
                        psrc.at[sidx.at[j]], av.at[pl.ds(j * IDX_ROW, IDX_ROW)], sem))
                    cps.append(pltpu.async_copy(
                        pdst.at[didx.at[j]], bv.at[pl.ds(j * IDX_ROW, IDX_ROW)], sem))
                for cp in cps:
                    cp.wait()
                e0 = row0 * IDX_ROW
                pltpu.sync_copy(av, a_out.at[pl.ds(e0, CHUNK)])
                pltpu.sync_copy(bv, b_out.at[pl.ds(e0, CHUNK)])

            return carry

        lax.fori_loop(0, trips, body, 0)

    return gather


def _make_scatter(E_pad):
    rows_total = E_pad // IDX_ROW
    total_chunks = rows_total // ROWS_PER_CHUNK
    trips = -(-total_chunks // NS)

    @functools.partial(
        pl.kernel,
        out_type=jax.ShapeDtypeStruct((NC * HALF, H), jnp.float32),
        mesh=_mesh(),
        compiler_params=pltpu.CompilerParams(use_tc_tiling_on_sc=False),
        scratch_types=[
            pltpu.VMEM((ROWS_PER_CHUNK, IDX_ROW), jnp.int32),
            pltpu.VMEM((CHUNK, H), jnp.float32),
            pltpu.VMEM_SHARED((AGG_ROWS, H), jnp.float32),
            pltpu.SemaphoreType.DMA,
        ],
    )
    def scatter(msg, lidx2d, zeros, out, didx, mv, aggr, sem):
        c = lax.axis_index("c")
        s = lax.axis_index("s")

        pltpu.sync_copy(zeros, mv)
        for t in range(2):
            pltpu.sync_copy(mv.at[pl.ds(0, SLAB // 2)],
                            aggr.at[pl.ds(s * SLAB + t * (SLAB // 2), SLAB // 2)])

        @pl.when(s == 0)
        def _():
            pltpu.sync_copy(mv.at[pl.ds(0, IDX_ROW)],
                            aggr.at[pl.ds(HALF, IDX_ROW)])

        plsc.subcore_barrier()

        def body(g, carry):
            ck = s + g * NS

            @pl.when(ck < total_chunks)
            def _():
                row0 = ck * ROWS_PER_CHUNK
                pltpu.sync_copy(lidx2d.at[pl.ds(c * rows_total + row0,
                                                ROWS_PER_CHUNK)], didx)
                pltpu.sync_copy(msg.at[pl.ds(row0 * IDX_ROW, CHUNK)], mv)
                for j in range(ROWS_PER_CHUNK):
                    pltpu.sync_copy(mv.at[pl.ds(j * IDX_ROW, IDX_ROW)],
                                    aggr.at[didx.at[j]], add=True)

            return carry

        lax.fori_loop(0, trips, body, 0)
        plsc.subcore_barrier()

        pltpu.sync_copy(aggr.at[pl.ds(s * SLAB, SLAB)],
                        out.at[pl.ds(c * HALF + s * SLAB, SLAB)])

    return scatter


def _node_proj_body(xr, w1a_r, w1b_r, b1_r, ps_r, pd_r):
    xb = xr[...]
    ps_r[...] = jnp.dot(xb, w1a_r[...], preferred_element_type=jnp.float32)
    pd_r[...] = jnp.dot(xb, w1b_r[...], preferred_element_type=jnp.float32) + b1_r[...]


def _edge_mlp_body(ar, br, er, dr_idx, w1c_r, w2_r, b2_r, w3_r, b3_r, w4_r,
                   b4_r, mr, lr):
    t = ar[...] + br[...]
    e3 = er[...]
    wc = w1c_r[...]
    for j in range(8):
        t = t + jnp.dot(e3[:, j, :], wc[3 * j:3 * j + 3, :],
                        preferred_element_type=jnp.float32)
    h = jnp.tanh(t)
    h = jnp.tanh(jnp.dot(h, w2_r[...], preferred_element_type=jnp.float32) + b2_r[...])
    h = jnp.tanh(jnp.dot(h, w3_r[...], preferred_element_type=jnp.float32) + b3_r[...])
    mr[...] = jnp.dot(h, w4_r[...], preferred_element_type=jnp.float32) + b4_r[...]
    d = dr_idx[...]
    lr[0, ...] = jnp.where(d < HALF, d, HALF)
    d1 = d - HALF
    lr[1, ...] = jnp.where(d1 >= 0, d1, HALF)


def _update_body(xr, p0_r, u1x_r, u1a_r, c1_r, u2_r, c2_r, u3_r, c3_r, dr):
    aggr = p0_r[...]
    u = jnp.tanh(jnp.dot(xr[...], u1x_r[...], preferred_element_type=jnp.float32)
                 + jnp.dot(aggr, u1a_r[...], preferred_element_type=jnp.float32)
                 + c1_r[...])
    u = jnp.tanh(jnp.dot(u, u2_r[...], preferred_element_type=jnp.float32) + c2_r[...])
    dr[...] = jnp.dot(u, u3_r[...], preferred_element_type=jnp.float32) + c3_r[...]


def _full_spec(shape):
    return pl.BlockSpec(shape, lambda i: (0,) * len(shape))


def kernel(x, edge_index, edge_attr, W1, b1, W2, b2, W3, b3, W4, b4,
           U1, c1, U2, c2, U3, c3):
    E = edge_index.shape[1]
    assert E % CHUNK == 0, "edge count must be a multiple of 2048"
    E_pad = E

    src2d = edge_index[0].reshape(-1, IDX_ROW)
    dst2d = edge_index[1].reshape(-1, IDX_ROW)

    eye8 = jnp.eye(8, dtype=jnp.float32)
    kron = lambda w: jnp.kron(eye8, w)
    tile8 = lambda b: jnp.tile(b, 8).reshape(1, 8 * b.shape[0])

    NP = N_NODES // 8
    psrc, pdst = pl.pallas_call(
        _node_proj_body,
        grid=(1,),
        in_specs=[
            pl.BlockSpec((NP, 8 * NF), lambda i: (0, 0)),
            _full_spec((8 * NF, 8 * H)), _full_spec((8 * NF, 8 * H)),
            _full_spec((1, 8 * H)),
        ],
        out_specs=[pl.BlockSpec((NP, 8 * H), lambda i: (0, 0)),
                   pl.BlockSpec((NP, 8 * H), lambda i: (0, 0))],
        out_shape=[jax.ShapeDtypeStruct((NP, 8 * H), jnp.float32),
                   jax.ShapeDtypeStruct((NP, 8 * H), jnp.float32)],
    )(x.reshape(NP, 8 * NF), kron(W1[NF:2 * NF]), kron(W1[:NF]), tile8(b1))

    a_g, b_g = _make_gather(E_pad)(psrc.reshape(N_NODES, H),
                                   pdst.reshape(N_NODES, H), src2d, dst2d)

    EP = E_pad // 8
    RE8 = 3200
    ge = EP // RE8
    RROW = 8 * RE8 // IDX_ROW
    rows_total = E_pad // IDX_ROW
    msg, lidx = pl.pallas_call(
        _edge_mlp_body,
        grid=(ge,),
        in_specs=[
            pl.BlockSpec((RE8, 8 * H), lambda i: (i, 0)),
            pl.BlockSpec((RE8, 8 * H), lambda i: (i, 0)),
            pl.BlockSpec((RE8, 8, 3), lambda i: (i, 0, 0)),
            pl.BlockSpec((RROW, IDX_ROW), lambda i: (i, 0)),
            _full_spec((24, 8 * H)),
            _full_spec((8 * H, 8 * H)), _full_spec((1, 8 * H)),
            _full_spec((8 * H, 8 * H)), _full_spec((1, 8 * H)),
            _full_spec((8 * H, 8 * H)), _full_spec((1, 8 * H)),
        ],
        out_specs=[pl.BlockSpec((RE8, 8 * H), lambda i: (i, 0)),
                   pl.BlockSpec((NC, RROW, IDX_ROW), lambda i: (0, i, 0))],
        out_shape=[jax.ShapeDtypeStruct((EP, 8 * H), jnp.float32),
                   jax.ShapeDtypeStruct((NC, rows_total, IDX_ROW), jnp.int32)],
    )(a_g.reshape(EP, 8 * H), b_g.reshape(EP, 8 * H),
      edge_attr.reshape(EP, 8, 3), dst2d, kron(W1[2 * NF:]), kron(W2),
      tile8(b2), kron(W3), tile8(b3), kron(W4), tile8(b4))
    msg = msg.reshape(E_pad, H)

    zeros = jnp.zeros((CHUNK, H), jnp.float32)
    aggr = _make_scatter(E_pad)(msg, lidx.reshape(NC * rows_total, IDX_ROW),
                                zeros)

    RN = 2000
    gn = N_NODES // RN
    delta = pl.pallas_call(
        _update_body,
        grid=(gn,),
        in_specs=[
            pl.BlockSpec((RN, NF), lambda i: (i, 0)),
            pl.BlockSpec((RN, H), lambda i: (i, 0)),
            _full_spec((NF, H)), _full_spec((H, H)), _full_spec((1, H)),
            _full_spec((H, H)), _full_spec((1, H)),
            _full_spec((H, 3)), _full_spec((1, 3)),
        ],
        out_specs=pl.BlockSpec((RN, 3), lambda i: (i, 0)),
        out_shape=jax.ShapeDtypeStruct((N_NODES, 3), jnp.float32),
    )(x, aggr, U1[:NF], U1[NF:],
      c1.reshape(1, H), U2, c2.reshape(1, H), U3, c3.reshape(1, 3))

    return delta


# speedup vs baseline: 9.5173x; 1.6404x over previous
import functools

import jax
import jax.numpy as jnp
from jax import lax
from jax.experimental import pallas as pl
from jax.experimental.pallas import tpu as pltpu
from jax.experimental.pallas import tpu_sc as plsc

N_NODES = 100000
NF = 6
H = 16

NC = 2
NS = 16
NW = NC * NS

IDX_ROW = 128
ROWS_PER_CHUNK = 16
CHUNK = IDX_ROW * ROWS_PER_CHUNK
EDGES_PER_SUPER = NW * CHUNK

HALF = 50176
SLAB = HALF // NS
AGG_ROWS = HALF + IDX_ROW

_mesh = lambda: plsc.VectorSubcoreMesh(core_axis_name="c", subcore_axis_name="s")


def _make_gather(E_pad):
    rows_total = E_pad // IDX_ROW
    total_chunks = rows_total // ROWS_PER_CHUNK
    trips = -(-total_chunks // NW)

    @functools.partial(
        pl.kernel,
        out_type=(jax.ShapeDtypeStruct((E_pad, H), jnp.float32),
                  jax.ShapeDtypeStruct((E_pad, H), jnp.float32)),
        mesh=_mesh(),
        compiler_params=pltpu.CompilerParams(use_tc_tiling_on_sc=False),
        scratch_types=[
            pltpu.VMEM((ROWS_PER_CHUNK, IDX_ROW), jnp.int32),
            pltpu.VMEM((ROWS_PER_CHUNK, IDX_ROW), jnp.int32),
            pltpu.VMEM((CHUNK, H), jnp.float32),
            pltpu.VMEM((CHUNK, H), jnp.float32),
            pltpu.SemaphoreType.DMA,
        ],
    )
    def gather(psrc, pdst, src2d, dst2d, a_out, b_out, sidx, didx, av, bv, sem):
        wid = lax.axis_index("s") * NC + lax.axis_index("c")

        def body(g, carry):
            ck = wid + g * NW

            @pl.when(ck < total_chunks)
            def _():
                row0 = ck * ROWS_PER_CHUNK
                pltpu.sync_copy(src2d.at[pl.ds(row0, ROWS_PER_CHUNK)], sidx)
                pltpu.sync_copy(dst2d.at[pl.ds(row0, ROWS_PER_CHUNK)], didx)
                cps = []
                for j in range(ROWS_PER_CHUNK):
                    cps.append(pltpu.async_copy(
                        psrc.at[sidx.at[j]], av.at[pl.ds(j * IDX_ROW, IDX_ROW)], sem))
                    cps.append(pltpu.async_copy(
                        pdst.at[didx.at[j]], bv.at[pl.ds(j * IDX_ROW, IDX_ROW)], sem))
                for cp in cps:
                    cp.wait()
                e0 = row0 * IDX_ROW
                pltpu.sync_copy(av, a_out.at[pl.ds(e0, CHUNK)])
                pltpu.sync_copy(bv, b_out.at[pl.ds(e0, CHUNK)])

            return carry

        lax.fori_loop(0, trips, body, 0)

    return gather


def _make_scatter(E_pad):
    rows_total = E_pad // IDX_ROW
    total_chunks = rows_total // ROWS_PER_CHUNK
    trips = -(-total_chunks // NS)

    @functools.partial(
        pl.kernel,
        out_type=jax.ShapeDtypeStruct((NC * HALF, H), jnp.float32),
        mesh=_mesh(),
        compiler_params=pltpu.CompilerParams(use_tc_tiling_on_sc=False),
        scratch_types=[
            pltpu.VMEM((ROWS_PER_CHUNK, IDX_ROW), jnp.int32),
            pltpu.VMEM((CHUNK, H), jnp.float32),
            pltpu.VMEM_SHARED((AGG_ROWS, H), jnp.float32),
            pltpu.SemaphoreType.DMA,
        ],
    )
    def scatter(msg, lidx2d, zeros, out, didx, mv, aggr, sem):
        c = lax.axis_index("c")
        s = lax.axis_index("s")

        pltpu.sync_copy(zeros, mv)
        for t in range(2):
            pltpu.sync_copy(mv.at[pl.ds(0, SLAB // 2)],
                            aggr.at[pl.ds(s * SLAB + t * (SLAB // 2), SLAB // 2)])

        @pl.when(s == 0)
        def _():
            pltpu.sync_copy(mv.at[pl.ds(0, IDX_ROW)],
                            aggr.at[pl.ds(HALF, IDX_ROW)])

        plsc.subcore_barrier()

        def body(g, carry):
            ck = s + g * NS

            @pl.when(ck < total_chunks)
            def _():
                row0 = ck * ROWS_PER_CHUNK
                pltpu.sync_copy(lidx2d.at[pl.ds(c * rows_total + row0,
                                                ROWS_PER_CHUNK)], didx)
                pltpu.sync_copy(msg.at[pl.ds(row0 * IDX_ROW, CHUNK)], mv)
                for j in range(ROWS_PER_CHUNK):
                    pltpu.sync_copy(mv.at[pl.ds(j * IDX_ROW, IDX_ROW)],
                                    aggr.at[didx.at[j]], add=True)

            return carry

        lax.fori_loop(0, trips, body, 0)
        plsc.subcore_barrier()

        pltpu.sync_copy(aggr.at[pl.ds(s * SLAB, SLAB)],
                        out.at[pl.ds(c * HALF + s * SLAB, SLAB)])

    return scatter


def _node_proj_body(xr, w1a_r, w1b_r, b1_r, ps_r, pd_r):
    xb = xr[...]
    ps_r[...] = jnp.dot(xb, w1a_r[...], preferred_element_type=jnp.float32)
    pd_r[...] = jnp.dot(xb, w1b_r[...], preferred_element_type=jnp.float32) + b1_r[...]


def _edge_mlp_body(ar, br, er, dr_idx, w1c_r, w2_r, b2_r, w3_r, b3_r, w4_r,
                   b4_r, mr, lr):
    t = ar[...] + br[...]
    ee = er[...]
    e3 = ee.reshape(ee.shape[0] // 8, 8, 3)
    wc = w1c_r[...]
    for j in range(8):
        t = t + jnp.dot(e3[:, j, :], wc[3 * j:3 * j + 3, :],
                        preferred_element_type=jnp.float32)
    h = jnp.tanh(t)
    h = jnp.tanh(jnp.dot(h, w2_r[...], preferred_element_type=jnp.float32) + b2_r[...])
    h = jnp.tanh(jnp.dot(h, w3_r[...], preferred_element_type=jnp.float32) + b3_r[...])
    mr[...] = jnp.dot(h, w4_r[...], preferred_element_type=jnp.float32) + b4_r[...]
    d = dr_idx[...]
    lr[0, ...] = jnp.where(d < HALF, d, HALF)
    d1 = d - HALF
    lr[1, ...] = jnp.where(d1 >= 0, d1, HALF)


def _update_body(xr, p0_r, u1x_r, u1a_r, c1_r, u2_r, c2_r, u3_r, c3_r, dr):
    aggr = p0_r[...]
    u = jnp.tanh(jnp.dot(xr[...], u1x_r[...], preferred_element_type=jnp.float32)
                 + jnp.dot(aggr, u1a_r[...], preferred_element_type=jnp.float32)
                 + c1_r[...])
    u = jnp.tanh(jnp.dot(u, u2_r[...], preferred_element_type=jnp.float32) + c2_r[...])
    dr[...] = jnp.dot(u, u3_r[...], preferred_element_type=jnp.float32) + c3_r[...]


def _full_spec(shape):
    return pl.BlockSpec(shape, lambda i: (0,) * len(shape))


def kernel(x, edge_index, edge_attr, W1, b1, W2, b2, W3, b3, W4, b4,
           U1, c1, U2, c2, U3, c3):
    E = edge_index.shape[1]
    assert E % CHUNK == 0, "edge count must be a multiple of 2048"
    E_pad = E

    src2d = edge_index[0].reshape(-1, IDX_ROW)
    dst2d = edge_index[1].reshape(-1, IDX_ROW)

    eye8 = jnp.eye(8, dtype=jnp.float32)
    kron = lambda w: jnp.kron(eye8, w)
    tile8 = lambda b: jnp.tile(b, 8).reshape(1, 8 * b.shape[0])

    NP = N_NODES // 8
    psrc, pdst = pl.pallas_call(
        _node_proj_body,
        grid=(1,),
        in_specs=[
            pl.BlockSpec((NP, 8 * NF), lambda i: (0, 0)),
            _full_spec((8 * NF, 8 * H)), _full_spec((8 * NF, 8 * H)),
            _full_spec((1, 8 * H)),
        ],
        out_specs=[pl.BlockSpec((NP, 8 * H), lambda i: (0, 0)),
                   pl.BlockSpec((NP, 8 * H), lambda i: (0, 0))],
        out_shape=[jax.ShapeDtypeStruct((NP, 8 * H), jnp.float32),
                   jax.ShapeDtypeStruct((NP, 8 * H), jnp.float32)],
    )(x.reshape(NP, 8 * NF), kron(W1[NF:2 * NF]), kron(W1[:NF]), tile8(b1))

    a_g, b_g = _make_gather(E_pad)(psrc.reshape(N_NODES, H),
                                   pdst.reshape(N_NODES, H), src2d, dst2d)

    EP = E_pad // 8
    RE8 = 3200
    ge = EP // RE8
    RROW = 8 * RE8 // IDX_ROW
    rows_total = E_pad // IDX_ROW
    msg, lidx = pl.pallas_call(
        _edge_mlp_body,
        grid=(ge,),
        in_specs=[
            pl.BlockSpec((RE8, 8 * H), lambda i: (i, 0)),
            pl.BlockSpec((RE8, 8 * H), lambda i: (i, 0)),
            pl.BlockSpec((8 * RE8, 3), lambda i: (i, 0)),
            pl.BlockSpec((RROW, IDX_ROW), lambda i: (i, 0)),
            _full_spec((24, 8 * H)),
            _full_spec((8 * H, 8 * H)), _full_spec((1, 8 * H)),
            _full_spec((8 * H, 8 * H)), _full_spec((1, 8 * H)),
            _full_spec((8 * H, 8 * H)), _full_spec((1, 8 * H)),
        ],
        out_specs=[pl.BlockSpec((RE8, 8 * H), lambda i: (i, 0)),
                   pl.BlockSpec((NC, RROW, IDX_ROW), lambda i: (0, i, 0))],
        out_shape=[jax.ShapeDtypeStruct((EP, 8 * H), jnp.float32),
                   jax.ShapeDtypeStruct((NC, rows_total, IDX_ROW), jnp.int32)],
    )(a_g.reshape(EP, 8 * H), b_g.reshape(EP, 8 * H),
      edge_attr, dst2d, kron(W1[2 * NF:]), kron(W2),
      tile8(b2), kron(W3), tile8(b3), kron(W4), tile8(b4))
    msg = msg.reshape(E_pad, H)

    zeros = jnp.zeros((CHUNK, H), jnp.float32)
    aggr = _make_scatter(E_pad)(msg, lidx.reshape(NC * rows_total, IDX_ROW),
                                zeros)

    RN = 2000
    gn = N_NODES // RN
    delta = pl.pallas_call(
        _update_body,
        grid=(gn,),
        in_specs=[
            pl.BlockSpec((RN, NF), lambda i: (i, 0)),
            pl.BlockSpec((RN, H), lambda i: (i, 0)),
            _full_spec((NF, H)), _full_spec((H, H)), _full_spec((1, H)),
            _full_spec((H, H)), _full_spec((1, H)),
            _full_spec((H, 3)), _full_spec((1, 3)),
        ],
        out_specs=pl.BlockSpec((RN, 3), lambda i: (i, 0)),
        out_shape=jax.ShapeDtypeStruct((N_NODES, 3), jnp.float32),
    )(x, aggr, U1[:NF], U1[NF:],
      c1.reshape(1, H), U2, c2.reshape(1, H), U3, c3.reshape(1, 3))

    return delta
